# baseline (speedup 1.0000x reference)
import functools
import math

import jax
import jax.numpy as jnp
from jax import lax
from jax.experimental import pallas as pl
from jax.experimental.pallas import tpu as pltpu
from jax.experimental.pallas import tpu_sc as plsc

_NC = 2
_NS = 16
_NW = _NC * _NS


@functools.partial(jax.jit, static_argnums=(2,))
def _emb_lookup(tpad, x, scale):
    b0, s_dim = x.shape
    d = tpad.shape[1] // 2
    rows_w = b0 // _NW
    n_chunks = rows_w
    n_pairs = n_chunks // 2
    ch = s_dim
    sub = ((0, 104), (104, 96))

    mesh = plsc.VectorSubcoreMesh(core_axis_name="c", subcore_axis_name="s")

    @functools.partial(
        pl.kernel,
        mesh=mesh,
        out_type=jax.ShapeDtypeStruct((b0, s_dim, d), jnp.float32),
        scratch_types=[
            pltpu.VMEM((rows_w, s_dim), jnp.int32),
            pltpu.VMEM((2, ch, 2 * d), jnp.float32),
            pltpu.VMEM((1, ch, d), jnp.float32),
            pltpu.SemaphoreType.DMA,
            pltpu.SemaphoreType.DMA,
        ],
        compiler_params=pltpu.CompilerParams(
            use_tc_tiling_on_sc=False, needs_layout_passes=False),
    )
    def k(tpad_hbm, x_hbm, out_hbm, xs, gbuf, obuf, sem0, sem1):
        wid = lax.axis_index("s") * _NC + lax.axis_index("c")
        sems = (sem0, sem1)

        pltpu.sync_copy(x_hbm.at[pl.ds(wid * rows_w, rows_w)], xs)

        def batches(g, b):
            for (c0, w) in sub:
                yield (xs.at[g, pl.ds(c0, w)],
                       gbuf.at[b].at[pl.ds(c0, w)])

        def fire(g, b):
            for idx_ref, dst in batches(g, b):
                pltpu.async_copy(tpad_hbm.at[idx_ref], dst, sems[b])

        def process(g, b):
            for idx_ref, dst in batches(g, b):
                pltpu.make_async_copy(tpad_hbm.at[idx_ref], dst,
                                      sems[b]).wait()

            def mul_body(s, c2):
                for l in range(d // 16):
                    obuf[0, s, pl.ds(l * 16, 16)] = (
                        gbuf[b, s, pl.ds(l * 16, 16)] * scale)
                return c2

            lax.fori_loop(0, ch, mul_body, 0)

            pltpu.sync_copy(obuf, out_hbm.at[pl.ds(wid * rows_w + g, 1)])

        fire(0, 0)

        def pair_body(gp, carry):
            g0 = 2 * gp
            fire(g0 + 1, 1)
            process(g0, 0)

            @pl.when(gp + 1 < n_pairs)
            def _():
                fire(g0 + 2, 0)

            process(g0 + 1, 1)
            return carry

        lax.fori_loop(0, n_pairs, pair_body, 0)

    return k(tpad, x)


def kernel(x, table):
    d = table.shape[1]
    scale = float(math.sqrt(d))
    x = x if x.dtype == jnp.int32 else x.astype(jnp.int32)
    tpad = jnp.pad(table, ((0, 0), (0, 128 - d)))
    return _emb_lookup(tpad, x, scale)


# speedup vs baseline: 1.2749x; 1.2749x over previous
import functools
import math

import jax
import jax.numpy as jnp
from jax import lax
from jax.experimental import pallas as pl
from jax.experimental.pallas import tpu as pltpu
from jax.experimental.pallas import tpu_sc as plsc

_NC = 2
_NS = 16
_NW = _NC * _NS

_XR = 2


@functools.partial(jax.jit, static_argnums=(2,))
def _emb_lookup(table, x, scale):
    b0, s_dim = x.shape
    d = table.shape[1]
    rows_w = b0 // _NW
    n_chunks = rows_w // _XR
    n_pairs = n_chunks // 2
    ch = _XR * s_dim
    sub = ((0, 104), (104, 96))

    mesh = plsc.VectorSubcoreMesh(core_axis_name="c", subcore_axis_name="s")

    @functools.partial(
        pl.kernel,
        mesh=mesh,
        out_type=jax.ShapeDtypeStruct((b0, s_dim, d), jnp.float32),
        scratch_types=[
            pltpu.VMEM((rows_w, s_dim), jnp.int32),
            pltpu.VMEM((2, ch, d), jnp.float32),
            pltpu.VMEM((_XR, s_dim, d), jnp.float32),
            pltpu.SemaphoreType.DMA,
            pltpu.SemaphoreType.DMA,
        ],
        compiler_params=pltpu.CompilerParams(
            use_tc_tiling_on_sc=False, needs_layout_passes=False),
    )
    def k(table_hbm, x_hbm, out_hbm, xs, gbuf, obuf, sem0, sem1):
        wid = lax.axis_index("s") * _NC + lax.axis_index("c")
        sems = (sem0, sem1)

        pltpu.sync_copy(x_hbm.at[pl.ds(wid * rows_w, rows_w)], xs)

        def batches(g, b):
            for r in range(_XR):
                for (c0, w) in sub:
                    yield (xs.at[g * _XR + r, pl.ds(c0, w)],
                           gbuf.at[b].at[pl.ds(r * s_dim + c0, w)])

        def fire(g, b):
            for idx_ref, dst in batches(g, b):
                pltpu.async_copy(table_hbm.at[idx_ref], dst, sems[b])

        def process(g, b):
            for idx_ref, dst in batches(g, b):
                pltpu.make_async_copy(table_hbm.at[idx_ref], dst,
                                      sems[b]).wait()

            def mul_body(s, c2):
                for r in range(_XR):
                    for l in range(d // 16):
                        obuf[r, s, pl.ds(l * 16, 16)] = (
                            gbuf[b, r * s_dim + s, pl.ds(l * 16, 16)] * scale)
                return c2

            lax.fori_loop(0, s_dim, mul_body, 0)

            pltpu.sync_copy(obuf,
                            out_hbm.at[pl.ds(wid * rows_w + g * _XR, _XR)])

        fire(0, 0)

        def pair_body(gp, carry):
            g0 = 2 * gp
            fire(g0 + 1, 1)
            process(g0, 0)

            @pl.when(gp + 1 < n_pairs)
            def _():
                fire(g0 + 2, 0)

            process(g0 + 1, 1)
            return carry

        lax.fori_loop(0, n_pairs, pair_body, 0)

    return k(table, x)


def kernel(x, table):
    d = table.shape[1]
    scale = float(math.sqrt(d))
    x = x if x.dtype == jnp.int32 else x.astype(jnp.int32)
    return _emb_lookup(table, x, scale)


# speedup vs baseline: 1.2755x; 1.0004x over previous
import functools
import math

import jax
import jax.numpy as jnp
from jax import lax
from jax.experimental import pallas as pl
from jax.experimental.pallas import tpu as pltpu
from jax.experimental.pallas import tpu_sc as plsc

_NC = 2
_NS = 16
_NW = _NC * _NS

_XR = 2


@functools.partial(jax.jit, static_argnums=(2,))
def _emb_lookup(table, x, scale):
    b0, s_dim = x.shape
    d = table.shape[1]
    rows_w = b0 // _NW
    n_chunks = rows_w // _XR
    n_pairs = n_chunks // 2
    ch = _XR * s_dim
    if s_dim <= 128:
        sub = ((0, s_dim),)
    else:
        h = (((s_dim + 1) // 2) + 7) // 8 * 8
        sub = ((0, h), (h, s_dim - h))

    mesh = plsc.VectorSubcoreMesh(core_axis_name="c", subcore_axis_name="s")

    @functools.partial(
        pl.kernel,
        mesh=mesh,
        out_type=jax.ShapeDtypeStruct((b0, s_dim, d), jnp.float32),
        scratch_types=[
            pltpu.VMEM((rows_w, s_dim), jnp.int32),
            pltpu.VMEM((2, ch, d), jnp.float32),
            pltpu.VMEM((_XR, s_dim, d), jnp.float32),
            pltpu.SemaphoreType.DMA,
            pltpu.SemaphoreType.DMA,
        ],
        compiler_params=pltpu.CompilerParams(
            use_tc_tiling_on_sc=False, needs_layout_passes=False),
    )
    def k(table_hbm, x_hbm, out_hbm, xs, gbuf, obuf, sem0, sem1):
        wid = lax.axis_index("s") * _NC + lax.axis_index("c")
        sems = (sem0, sem1)

        pltpu.sync_copy(x_hbm.at[pl.ds(wid * rows_w, rows_w)], xs)

        def batches(g, b):
            for r in range(_XR):
                for (c0, w) in sub:
                    yield (xs.at[g * _XR + r, pl.ds(c0, w)],
                           gbuf.at[b].at[pl.ds(r * s_dim + c0, w)])

        def fire(g, b):
            for idx_ref, dst in batches(g, b):
                pltpu.async_copy(table_hbm.at[idx_ref], dst, sems[b])

        def process(g, b):
            for idx_ref, dst in batches(g, b):
                pltpu.make_async_copy(table_hbm.at[idx_ref], dst,
                                      sems[b]).wait()

            def mul_body(s, c2):
                for r in range(_XR):
                    for l in range(d // 16):
                        obuf[r, s, pl.ds(l * 16, 16)] = (
                            gbuf[b, r * s_dim + s, pl.ds(l * 16, 16)] * scale)
                return c2

            lax.fori_loop(0, s_dim, mul_body, 0)

            pltpu.sync_copy(obuf,
                            out_hbm.at[pl.ds(wid * rows_w + g * _XR, _XR)])

        fire(0, 0)

        def pair_body(gp, carry):
            g0 = 2 * gp
            fire(g0 + 1, 1)
            process(g0, 0)

            @pl.when(gp + 1 < n_pairs)
            def _():
                fire(g0 + 2, 0)

            process(g0 + 1, 1)
            return carry

        lax.fori_loop(0, n_pairs, pair_body, 0)

    return k(table, x)


def kernel(x, table):
    d = table.shape[1]
    scale = float(math.sqrt(d))
    x = x if x.dtype == jnp.int32 else x.astype(jnp.int32)
    return _emb_lookup(table, x, scale)
